# Initial kernel scaffold; baseline (speedup 1.0000x reference)
import jax
import jax.numpy as jnp
from jax.experimental import pallas as pl


def kernel(word_ids, W):
    raise NotImplementedError("write your pallas kernel here")


# speedup vs baseline: 2.5223x; 2.5223x over previous
import functools

import jax
import jax.numpy as jnp
from jax import lax
from jax.experimental import pallas as pl
from jax.experimental.pallas import tpu as pltpu
from jax.experimental.pallas import tpu_sc as plsc

NW = 32
CB = 2
LANES = 16


def _emb_mean_kernel(B, L, D, idx_hbm, table_hbm, out_hbm,
                     idx_v, rows0, rows1, acc_v, sem0, sem1):
    BPW = B // NW
    NCH = BPW // CB
    ROWS = CB * L
    NV = D // LANES
    inv = jnp.float32(1.0 / L)

    nc = plsc.get_sparse_core_info().num_cores
    wid = lax.axis_index("s") * nc + lax.axis_index("c")

    pltpu.sync_copy(idx_hbm.at[wid], idx_v)

    bufs = (rows0, rows1)
    sems = (sem0, sem1)

    def start(c, b):
        pltpu.async_copy(table_hbm.at[idx_v.at[c]], bufs[b], sems[b])

    def wait(b):
        pltpu.make_async_copy(table_hbm.at[idx_v.at[0]], bufs[b], sems[b]).wait()

    def accumulate(c, b):
        rows = bufs[b]
        for j in range(CB):
            regs = [jnp.zeros((LANES,), jnp.float32) for _ in range(NV)]
            for r in range(L):
                for k in range(NV):
                    regs[k] = regs[k] + rows[j * L + r, pl.ds(k * LANES, LANES)]
            for k in range(NV):
                acc_v[c * CB + j, pl.ds(k * LANES, LANES)] = regs[k] * inv

    start(0, 0)
    start(1, 1)

    def body(g, carry):
        for b in range(2):
            c = 2 * g + b
            wait(b)
            accumulate(c, b)

            @pl.when(c + 2 < NCH)
            def _():
                start(c + 2, b)
        return carry

    lax.fori_loop(0, NCH // 2, body, 0)

    pltpu.sync_copy(acc_v, out_hbm.at[pl.ds(wid * BPW, BPW)])


@functools.partial(jax.jit, static_argnames=("B", "L", "D"))
def _emb_mean(idx, W, B, L, D):
    BPW = B // NW
    NCH = BPW // CB
    ROWS = CB * L
    mesh = plsc.VectorSubcoreMesh(core_axis_name="c", subcore_axis_name="s")
    return pl.kernel(
        functools.partial(_emb_mean_kernel, B, L, D),
        out_type=jax.ShapeDtypeStruct((B, D), jnp.float32),
        mesh=mesh,
        compiler_params=pltpu.CompilerParams(use_tc_tiling_on_sc=False),
        scratch_types=[
            pltpu.VMEM((NCH, ROWS), jnp.int32),
            pltpu.VMEM((ROWS, D), jnp.float32),
            pltpu.VMEM((ROWS, D), jnp.float32),
            pltpu.VMEM((BPW, D), jnp.float32),
            pltpu.SemaphoreType.DMA,
            pltpu.SemaphoreType.DMA,
        ],
    )(idx, W)


def kernel(word_ids, W):
    B, L = word_ids.shape
    D = W.shape[1]
    BPW = B // NW
    assert B % NW == 0 and BPW % CB == 0 and D % LANES == 0
    idx = word_ids.astype(jnp.int32).reshape(NW, BPW // CB, CB * L)
    return _emb_mean(idx, W, B, L, D)
